# Initial kernel scaffold; baseline (speedup 1.0000x reference)
import jax
import jax.numpy as jnp
from jax.experimental import pallas as pl


def kernel(x, edge_index, batch, Wl0, bl0, Wr0, g0, b0, Wl1, bl1, Wr1, g1, b1, Wl2, bl2, Wr2, g2, b2, Wc, bc):
    raise NotImplementedError("write your pallas kernel here")


# speedup vs baseline: 5.1040x; 5.1040x over previous
import functools

import jax
import jax.numpy as jnp
from jax import lax
from jax.experimental import pallas as pl
from jax.experimental.pallas import tpu as pltpu
from jax.experimental.pallas import tpu_sc as plsc

N = 10000
E = 320000
D = 128
H = 64
C = 10
G = 64

NC = 2
NS = 16
NW = NC * NS
CHUNK = 128
K = 2
NBUF = 2
NPAD = 10240
STRIPE = NPAD // NS
EPW = CHUNK * K * NBUF
EP = ((E + NW * EPW - 1) // (NW * EPW)) * (NW * EPW)
GPW = EP // (NW * CHUNK)
NB = GPW // K
W0 = 80

_f32 = jnp.float32


def _make_segsum(W):
    mesh = plsc.VectorSubcoreMesh(core_axis_name="c", subcore_axis_name="s")

    @functools.partial(
        pl.kernel,
        out_type=jax.ShapeDtypeStruct((2 * NPAD, W), _f32),
        mesh=mesh,
        compiler_params=pltpu.CompilerParams(use_tc_tiling_on_sc=False),
        scratch_types=[
            pltpu.VMEM_SHARED((NPAD, W), _f32),
            pltpu.VMEM((GPW, CHUNK), jnp.int32),
            pltpu.VMEM((GPW, CHUNK), jnp.int32),
        ] + [pltpu.VMEM((CHUNK, W), _f32)] * (K * NBUF) + [
            pltpu.SemaphoreType.DMA,
            pltpu.SemaphoreType.DMA,
        ],
    )
    def seg(table, srcp, dstp, out, acc, idx_s, idx_d,
            r0, r1, r2, r3, sem0, sem1):
        c = lax.axis_index("c")
        s = lax.axis_index("s")
        wid = s * NC + c
        bufs = ((r0, r1), (r2, r3))
        sems = (sem0, sem1)

        pltpu.sync_copy(srcp.at[pl.ds(wid * GPW, GPW)], idx_s)
        pltpu.sync_copy(dstp.at[pl.ds(wid * GPW, GPW)], idx_d)

        def zrow(i, _):
            def zcol(j, _):
                r0[i, pl.ds(j * 16, 16)] = jnp.zeros((16,), _f32)
                return 0
            return lax.fori_loop(0, W // 16, zcol, 0)
        lax.fori_loop(0, CHUNK, zrow, 0)

        def zacc(k, _):
            b = pl.multiple_of(s * STRIPE + k * CHUNK, CHUNK)
            pltpu.sync_copy(r0, acc.at[pl.ds(b, CHUNK)])
            return 0
        lax.fori_loop(0, STRIPE // CHUNK, zacc, 0)
        plsc.subcore_barrier()

        def fire(b, p):
            for j in range(K):
                pltpu.async_copy(table.at[idx_s.at[b * K + j]],
                                 bufs[p][j], sems[p])

        def drain_scatter(b, p):
            for j in range(K):
                pltpu.make_async_copy(table.at[idx_s.at[b * K + j]],
                                      bufs[p][j], sems[p]).wait()
            for j in range(K):
                pltpu.sync_copy(bufs[p][j], acc.at[idx_d.at[b * K + j]],
                                add=True)

        fire(0, 0)

        def body(i, _):
            b0 = 2 * i
            fire(b0 + 1, 1)
            drain_scatter(b0, 0)

            @pl.when(b0 + 2 < NB)
            def _():
                fire(b0 + 2, 0)
            drain_scatter(b0 + 1, 1)
            return 0
        lax.fori_loop(0, NB // 2, body, 0)
        plsc.subcore_barrier()

        def cp(k, _):
            b = pl.multiple_of(s * STRIPE + k * CHUNK, CHUNK)
            pltpu.sync_copy(acc.at[pl.ds(b, CHUNK)], r0)
            ob = pl.multiple_of(c * NPAD + b, CHUNK)
            pltpu.sync_copy(r0, out.at[pl.ds(ob, CHUNK)])
            return 0
        lax.fori_loop(0, STRIPE // CHUNK, cp, 0)

    return seg


_segsum80 = _make_segsum(W0)
_segsum64 = _make_segsum(H)


def _tc0(x_ref, wl_ref, wr_ref, bl_ref, p_ref, r_ref):
    x = x_ref[...]
    p = jnp.dot(x, wl_ref[...], preferred_element_type=_f32)
    p_ref[...] = jnp.concatenate([p, jnp.ones((N, W0 - H), _f32)], axis=1)
    r_ref[...] = jnp.dot(x, wr_ref[...], preferred_element_type=_f32) + bl_ref[...]


def _bn_relu(z, g, b):
    mu = jnp.mean(z, axis=0, keepdims=True)
    zc = z - mu
    var = jnp.mean(zc * zc, axis=0, keepdims=True)
    return jnp.maximum(zc * lax.rsqrt(var + 1e-5) * g + b, 0.0)


def _tc1(parts_ref, r_ref, g_ref, b_ref, wl_ref, wr_ref, bl_ref,
         p_ref, ro_ref, invd_ref):
    parts = parts_ref[...]
    a = parts[:NPAD] + parts[NPAD:]
    agg = a[:N, :H]
    deg = a[:N, H:H + 1]
    invd = 1.0 / jnp.maximum(deg, 1.0)
    h = _bn_relu(agg * invd + r_ref[...], g_ref[...], b_ref[...])
    p_ref[...] = jnp.dot(h, wl_ref[...], preferred_element_type=_f32)
    ro_ref[...] = jnp.dot(h, wr_ref[...], preferred_element_type=_f32) + bl_ref[...]
    invd_ref[...] = invd


def _tc2(parts_ref, invd_ref, r_ref, g_ref, b_ref, wl_ref, wr_ref, bl_ref,
         p_ref, ro_ref):
    parts = parts_ref[...]
    agg = (parts[:NPAD] + parts[NPAD:])[:N]
    h = _bn_relu(agg * invd_ref[...] + r_ref[...], g_ref[...], b_ref[...])
    p_ref[...] = jnp.dot(h, wl_ref[...], preferred_element_type=_f32)
    ro_ref[...] = jnp.dot(h, wr_ref[...], preferred_element_type=_f32) + bl_ref[...]


def _tc3(parts_ref, invd_ref, r_ref, g_ref, b_ref, blane_ref, wc_ref, bc_ref,
         out_ref):
    parts = parts_ref[...]
    agg = (parts[:NPAD] + parts[NPAD:])[:N]
    h = _bn_relu(agg * invd_ref[...] + r_ref[...], g_ref[...], b_ref[...])
    oh = (jax.lax.broadcasted_iota(jnp.int32, (G, N), 0)
          == blane_ref[...]).astype(_f32)
    pooled = jnp.dot(oh, h, preferred_element_type=_f32)
    cnt = jnp.sum(oh, axis=1, keepdims=True)
    emb = pooled / jnp.maximum(cnt, 1.0)
    out_ref[...] = jnp.dot(emb, wc_ref[...], preferred_element_type=_f32) + bc_ref[...]


def kernel(x, edge_index, batch, Wl0, bl0, Wr0, g0, b0, Wl1, bl1, Wr1, g1, b1,
           Wl2, bl2, Wr2, g2, b2, Wc, bc):
    src = edge_index[0]
    dst = edge_index[1]
    pad = EP - E
    srcp = jnp.concatenate([src, jnp.zeros((pad,), jnp.int32)]).reshape(
        EP // CHUNK, CHUNK)
    dstp = jnp.concatenate([dst, jnp.full((pad,), N, jnp.int32)]).reshape(
        EP // CHUNK, CHUNK)
    blane = batch.reshape(1, N)
    r2 = lambda v: v.reshape(1, -1)

    p0, r0 = pl.pallas_call(
        _tc0,
        out_shape=[jax.ShapeDtypeStruct((N, W0), _f32),
                   jax.ShapeDtypeStruct((N, H), _f32)],
    )(x, Wl0, Wr0, r2(bl0))

    parts0 = _segsum80(p0, srcp, dstp)

    p1, r1, invd = pl.pallas_call(
        _tc1,
        out_shape=[jax.ShapeDtypeStruct((N, H), _f32),
                   jax.ShapeDtypeStruct((N, H), _f32),
                   jax.ShapeDtypeStruct((N, 1), _f32)],
    )(parts0, r0, r2(g0), r2(b0), Wl1, Wr1, r2(bl1))

    parts1 = _segsum64(p1, srcp, dstp)

    p2, r2b = pl.pallas_call(
        _tc2,
        out_shape=[jax.ShapeDtypeStruct((N, H), _f32),
                   jax.ShapeDtypeStruct((N, H), _f32)],
    )(parts1, invd, r1, r2(g1), r2(b1), Wl2, Wr2, r2(bl2))

    parts2 = _segsum64(p2, srcp, dstp)

    out = pl.pallas_call(
        _tc3,
        out_shape=jax.ShapeDtypeStruct((G, C), _f32),
    )(parts2, invd, r2b, r2(g2), r2(b2), blane, Wc, r2(bc))

    return out
